# baseline (speedup 1.0000x reference)
import jax
import jax.numpy as jnp
from jax.experimental import pallas as pl
from jax.experimental.pallas import tpu as pltpu

B = 16384
D = 32
H = 64
N = 1_000_000
LANES = 128
IPB = 32
CAT = IPB * LANES


def _body(uidx_ref, iidx_ref, *refs):
    uiv_ref, iiv_ref = refs[0], refs[1]
    t_refs = refs[2: 2 + 2 * IPB]
    w1u_ref, w1i_ref, b1_ref, w2_ref, b2_ref, o_ref = refs[2 + 2 * IPB:]

    l4 = jax.lax.broadcasted_iota(jnp.int32, (IPB, CAT), 1)
    li = jnp.bitwise_and(l4, LANES - 1)
    seg = 128 * jax.lax.broadcasted_iota(jnp.int32, (1, IPB), 1)

    def gather_rows(blocks, idxv):
        cat = jnp.concatenate([r[...] for r in blocks], axis=1)
        lane = jnp.bitwise_and(idxv, LANES - 1)
        p = jnp.transpose(lane + seg)
        base = jnp.transpose(idxv - lane)
        e = jnp.where((l4 == p) & (base + li < N), 1.0, 0.0)
        return jax.lax.dot_general(e, cat, (((1,), (1,)), ((), ())),
                                   preferred_element_type=jnp.float32)

    u = gather_rows(t_refs[:IPB], uiv_ref[0])
    i = gather_rows(t_refs[IPB:], iiv_ref[0])
    h = jnp.dot(u, w1u_ref[...], preferred_element_type=jnp.float32)
    h = h + jnp.dot(i, w1i_ref[...], preferred_element_type=jnp.float32)
    h = jnp.maximum(h + b1_ref[...], 0.0)
    o_ref[...] = jnp.dot(h, w2_ref[...],
                         preferred_element_type=jnp.float32) + b2_ref[...]


def kernel(user, item, user_table, item_table, W1, b1, W2, b2):
    uT = user_table.T
    iT = item_table.T
    user = user.astype(jnp.int32)
    item = item.astype(jnp.int32)

    def tile_spec(scalar_slot, k):
        def index_map(g, u_idx, i_idx):
            idx = (u_idx if scalar_slot == 0 else i_idx)[g * IPB + k]
            return (0, idx // LANES)
        return pl.BlockSpec((D, LANES), index_map)

    in_specs = (
        [
            pl.BlockSpec((1, 1, IPB), lambda g, u, i: (g, 0, 0)),
            pl.BlockSpec((1, 1, IPB), lambda g, u, i: (g, 0, 0)),
        ]
        + [tile_spec(0, k) for k in range(IPB)]
        + [tile_spec(1, k) for k in range(IPB)]
        + [
            pl.BlockSpec((D, H), lambda g, u, i: (0, 0)),
            pl.BlockSpec((D, H), lambda g, u, i: (0, 0)),
            pl.BlockSpec((1, H), lambda g, u, i: (0, 0)),
            pl.BlockSpec((H, 1), lambda g, u, i: (0, 0)),
            pl.BlockSpec((1, 1), lambda g, u, i: (0, 0)),
        ]
    )
    grid_spec = pltpu.PrefetchScalarGridSpec(
        num_scalar_prefetch=2,
        grid=(B // IPB,),
        in_specs=in_specs,
        out_specs=pl.BlockSpec((IPB, 1), lambda g, u, i: (g, 0)),
    )
    out = pl.pallas_call(
        _body,
        grid_spec=grid_spec,
        out_shape=jax.ShapeDtypeStruct((B, 1), jnp.float32),
    )(user, item,
      user.reshape(B // IPB, 1, IPB), item.reshape(B // IPB, 1, IPB),
      *([uT] * IPB), *([iT] * IPB),
      W1[:D], W1[D:], b1.reshape(1, H), W2, b2.reshape(1, 1))
    return out[:, 0]


# speedup vs baseline: 1.0155x; 1.0155x over previous
import jax
import jax.numpy as jnp
from jax.experimental import pallas as pl
from jax.experimental.pallas import tpu as pltpu

B = 16384
D = 32
H = 64
N = 1_000_000
LANES = 128
IPB = 64
CAT = IPB * LANES


def _body(uidx_ref, iidx_ref, *refs):
    uiv_ref, iiv_ref = refs[0], refs[1]
    t_refs = refs[2: 2 + 2 * IPB]
    w1u_ref, w1i_ref, b1_ref, w2_ref, b2_ref, o_ref = refs[2 + 2 * IPB:]

    l4 = jax.lax.broadcasted_iota(jnp.int32, (IPB, CAT), 1)
    li = jnp.bitwise_and(l4, LANES - 1)
    seg = 128 * jax.lax.broadcasted_iota(jnp.int32, (1, IPB), 1)

    def gather_rows(blocks, idxv):
        cat = jnp.concatenate([r[...] for r in blocks], axis=1)
        lane = jnp.bitwise_and(idxv, LANES - 1)
        p = jnp.transpose(lane + seg)
        base = jnp.transpose(idxv - lane)
        e = jnp.where((l4 == p) & (base + li < N), 1.0, 0.0)
        return jax.lax.dot_general(e, cat, (((1,), (1,)), ((), ())),
                                   preferred_element_type=jnp.float32)

    u = gather_rows(t_refs[:IPB], uiv_ref[0])
    i = gather_rows(t_refs[IPB:], iiv_ref[0])
    h = jnp.dot(u, w1u_ref[...], preferred_element_type=jnp.float32)
    h = h + jnp.dot(i, w1i_ref[...], preferred_element_type=jnp.float32)
    h = jnp.maximum(h + b1_ref[...], 0.0)
    o_ref[...] = jnp.dot(h, w2_ref[...],
                         preferred_element_type=jnp.float32) + b2_ref[...]


def kernel(user, item, user_table, item_table, W1, b1, W2, b2):
    uT = user_table.T
    iT = item_table.T
    user = user.astype(jnp.int32)
    item = item.astype(jnp.int32)

    def tile_spec(scalar_slot, k):
        def index_map(g, u_idx, i_idx):
            idx = (u_idx if scalar_slot == 0 else i_idx)[g * IPB + k]
            return (0, idx // LANES)
        return pl.BlockSpec((D, LANES), index_map)

    in_specs = (
        [
            pl.BlockSpec((1, 1, IPB), lambda g, u, i: (g, 0, 0)),
            pl.BlockSpec((1, 1, IPB), lambda g, u, i: (g, 0, 0)),
        ]
        + [tile_spec(0, k) for k in range(IPB)]
        + [tile_spec(1, k) for k in range(IPB)]
        + [
            pl.BlockSpec((D, H), lambda g, u, i: (0, 0)),
            pl.BlockSpec((D, H), lambda g, u, i: (0, 0)),
            pl.BlockSpec((1, H), lambda g, u, i: (0, 0)),
            pl.BlockSpec((H, 1), lambda g, u, i: (0, 0)),
            pl.BlockSpec((1, 1), lambda g, u, i: (0, 0)),
        ]
    )
    grid_spec = pltpu.PrefetchScalarGridSpec(
        num_scalar_prefetch=2,
        grid=(B // IPB,),
        in_specs=in_specs,
        out_specs=pl.BlockSpec((IPB, 1), lambda g, u, i: (g, 0)),
    )
    out = pl.pallas_call(
        _body,
        grid_spec=grid_spec,
        out_shape=jax.ShapeDtypeStruct((B, 1), jnp.float32),
    )(user, item,
      user.reshape(B // IPB, 1, IPB), item.reshape(B // IPB, 1, IPB),
      *([uT] * IPB), *([iT] * IPB),
      W1[:D], W1[D:], b1.reshape(1, H), W2, b2.reshape(1, 1))
    return out[:, 0]
